# baseline (speedup 1.0000x reference)
import functools

import jax
import jax.numpy as jnp
from jax import lax
from jax.experimental import pallas as pl
from jax.experimental.pallas import tpu as pltpu

_NSWEEP = 12
_LAG = 4


def _pro_chunk_kernel(
    steps_ref,
    init_ref,
    l_ref,
    lrows_ref,
    lam_ref,
    w1_ref,
    w2_ref,
    u2d_ref,
    w3_ref,
    wi_ref,
    wb_ref,
    b_ref,
    out_ref,
    state_out_ref,
    state_s,
    e_s,
    eb_s,
    pk_s,
    st_s,
    *,
    n_stim, n_resp, n_ro, n_delay, hidden,
    chunk, last_chunk, last_local, lam_chunk, lam_last,
    dt, beta, psi, phi, rho, response_threshold,
):
    f32 = jnp.float32
    bf16 = jnp.bfloat16
    S, R, RO, H = n_stim, n_resp, n_ro, hidden
    DS = n_delay * S
    CH = chunk
    cid = pl.program_id(0)

    @pl.when(cid == 0)
    def _():
        state_s[...] = init_ref[...]

    stim = steps_ref[:, 0:S]
    noise = steps_ref[:, S:S + R]

    OFF = 16
    for i in range(n_delay - 1):
        lo = (n_delay - 2 - i) * S
        row = state_s[:, lo:lo + S]
        e_s[OFF - (n_delay - 1) + i:OFF - (n_delay - 1) + i + 1, :] = row
        eb_s[OFF - (n_delay - 1) + i:OFF - (n_delay - 1) + i + 1, :] = row.astype(bf16)
    e_s[OFF:OFF + CH, :] = stim
    eb_s[OFF:OFF + CH, :] = stim.astype(bf16)
    d_mat_b = jnp.concatenate(
        [eb_s[OFF - d:OFF - d + CH, :] for d in range(n_delay)],
        axis=1)

    carry = state_s[:, DS:2 * DS]
    du = jnp.dot(d_mat_b, u2d_ref[...], preferred_element_type=f32)
    cu = jnp.dot(carry.astype(bf16), u2d_ref[...],
                 preferred_element_type=f32)
    temporal = (jnp.dot(l_ref[...], du, preferred_element_type=f32)
                + lam_ref[...] * cu)
    er_dot = jnp.dot(lrows_ref[...], d_mat_b,
                     preferred_element_type=f32)

    fc1_b = b_ref[:, 0:H]
    fc2_b = b_ref[:, H:H + RO]

    res1 = jnp.dot(stim, w1_ref[...], preferred_element_type=f32)
    ws_out = res1[:, 0:RO]
    wc_out = res1[:, RO:RO + R]
    h = jnp.maximum(res1[:, RO + R:RO + R + H] + fc1_b, 0.0)

    valence = jnp.dot(h, w2_ref[...], preferred_element_type=f32) + fc2_b
    ro_pred = ws_out * valence

    lhs3 = jnp.concatenate([ro_pred, jnp.maximum(ro_pred, 0.0)], axis=1)
    res3 = jnp.dot(lhs3, w3_ref[...], preferred_element_type=f32)

    exc = wc_out
    ctb = jnp.maximum(res3[:, R:2 * R], 0.0) - res3[:, 0:R]
    r_vec = exc + noise - 0.05 * ctb
    sm1 = 1.0 - exc - ctb

    G = 8
    W = CH // G
    K = 16
    P = K + W
    c0 = state_s[:, 2 * DS:2 * DS + R]
    w_i = wi_ref[...]

    @pl.when(cid == 0)
    def _():
        inh0 = jnp.dot(c0, w_i, preferred_element_type=f32)
        z0 = jnp.log(c0 / (1.0 - c0))
        st_s[:, 0:R] = jnp.broadcast_to(z0 + (c0 + 0.05) * inh0, (K, R))
        st_s[:, R:2 * R] = jnp.zeros((K, R), f32)

    stash_r = st_s[:, 0:R]
    stash_s = st_s[:, R:2 * R]
    cols_r, cols_s = [], []
    for g in range(G):
        lo = g * W - K
        if lo < 0:
            cols_r.append(jnp.concatenate(
                [stash_r[lo + K:K, :], r_vec[0:g * W + W, :]], axis=0))
            cols_s.append(jnp.concatenate(
                [stash_s[lo + K:K, :], sm1[0:g * W + W, :]], axis=0))
        else:
            cols_r.append(r_vec[lo:lo + P, :])
            cols_s.append(sm1[lo:lo + P, :])
    rp_pk = jnp.concatenate(cols_r, axis=1)
    sm_pk = jnp.concatenate(cols_s, axis=1)

    pk_s[0:1, :] = jnp.concatenate([c0] * G, axis=1)

    @pl.when(cid == 0)
    def _():
        pk_s[1:P + 1, :] = jnp.broadcast_to(
            jnp.concatenate([c0] * G, axis=1), (P, G * R))

    wb = wb_ref[...]
    inh_q = []
    for m in range(_NSWEEP):
        src = pk_s[0:P, :]
        if m <= _NSWEEP - 1 - _LAG or m == 0:
            inh_q.append(jnp.dot(src, wb, preferred_element_type=f32))
        inh = inh_q[max(0, m - _LAG)]
        pre = src * sm_pk + rp_pk - (src + 0.05) * inh
        pk_s[1:P + 1, :] = 0.5 * jnp.tanh(0.5 * pre) + 0.5

    traj = pk_s[K + 1:P + 1, :]
    c_all = jnp.concatenate(
        [traj[:, g * R:(g + 1) * R] for g in range(G)], axis=0)

    st_s[:, 0:R] = r_vec[CH - K:CH, :]
    st_s[:, R:2 * R] = sm1[CH - K:CH, :]

    out_ref[:, 0:R] = c_all
    out_ref[:, R:2 * R] = (c_all > response_threshold).astype(f32)
    out_ref[:, 2 * R:2 * R + RO] = ro_pred
    out_ref[:, 2 * R + RO:2 * R + 2 * RO] = temporal
    out_ref[:, 2 * R + 2 * RO:] = jnp.zeros(
        (CH, out_ref.shape[1] - (2 * R + 2 * RO)), f32)

    d_last = jnp.concatenate(
        [e_s[OFF - d + CH - 1:OFF - d + CH, :]
         for d in range(n_delay)], axis=1)
    state_s[:, 0:DS] = d_last
    state_s[:, DS:2 * DS] = er_dot[0:1, :] + lam_chunk * carry
    state_s[:, 2 * DS:2 * DS + R] = c_all[CH - 1:CH, :]

    @pl.when(cid == last_chunk)
    def _():
        state_out_ref[:, 0:DS] = jnp.concatenate(
            [e_s[OFF - d + last_local:OFF - d + last_local + 1, :]
             for d in range(n_delay)], axis=1)
        state_out_ref[:, DS:2 * DS] = er_dot[1:2, :] + lam_last * carry
        state_out_ref[:, 2 * DS:2 * DS + R] = c_all[last_local:last_local + 1, :]


def kernel(W_S, W_C, W_F, W_R, W_I, U, fc1_w, fc1_b, fc2_w, fc2_b,
           delay_chain, eligibility_trace, C, stimuli_seq, noise_seq):
    f32 = jnp.float32
    bf16 = jnp.bfloat16
    hp = dict(dt=0.1, beta=0.1, lambda_decay=0.95, psi=0.1, phi=0.1, rho=0.1,
              response_threshold=0.5)
    n_ro, n_stim = W_S.shape
    n_resp = W_C.shape[0]
    n_delay = delay_chain.shape[0]
    hidden = fc1_b.shape[0]
    S, R, RO, H = n_stim, n_resp, n_ro, hidden
    DS = n_delay * S
    state_len = 2 * DS + R
    T = int(stimuli_seq.shape[0])
    OUT_W = 128
    assert 2 * R + 2 * RO <= OUT_W

    max_chunk = 2048
    chunk = min(max_chunk, ((T + 7) // 8) * 8)
    t_pad = ((T + chunk - 1) // chunk) * chunk
    num_chunks = t_pad // chunk
    last_chunk = (T - 1) // chunk
    last_local = (T - 1) % chunk

    lam = float(hp["lambda_decay"])
    idx = jnp.arange(chunk)
    diff = idx[:, None] - idx[None, :]
    lam_mat = jnp.where(
        diff >= 0,
        jnp.power(jnp.float32(lam), jnp.maximum(diff, 0).astype(f32)),
        0.0).astype(f32)
    l_rows = jnp.zeros((8, chunk), f32)
    l_rows = l_rows.at[0].set(lam_mat[chunk - 1])
    l_rows = l_rows.at[1].set(lam_mat[last_local])
    l_rows = l_rows.astype(bf16)
    lam_pows = jnp.power(jnp.float32(lam),
                         (idx + 1).astype(f32)).reshape(chunk, 1)

    bdt_h = float(hp["beta"]) * float(hp["dt"])
    w1 = jnp.zeros((S, RO + R + H), f32)
    w1 = w1.at[:, :RO].set(W_S.T.astype(f32))
    w1 = w1.at[:, RO:RO + R].set((bdt_h * float(hp["rho"])) * W_C.T.astype(f32))
    w1 = w1.at[:, RO + R:].set(fc1_w.T.astype(f32))

    w2 = fc2_w.T.astype(f32)
    u2d = U.astype(f32).reshape(RO, DS).T.astype(bf16)

    w3 = jnp.zeros((2 * RO, 2 * R), f32)
    w3 = w3.at[:RO, :R].set(W_F.astype(f32))
    w3 = w3.at[RO:2 * RO, R:2 * R].set(W_R.T.astype(f32))
    w3 = (bdt_h * float(hp["phi"])) * w3

    bdt = float(hp["beta"]) * float(hp["dt"])
    w_i = (bdt * float(hp["psi"])) * W_I.astype(f32)
    wb = jnp.kron(jnp.eye(8, dtype=f32), w_i)
    bias = jnp.concatenate([fc1_b.astype(f32),
                            fc2_b.astype(f32)]).reshape(1, H + RO)

    stim = jnp.zeros((t_pad, S), f32).at[:T].set(
        stimuli_seq.reshape(T, S).astype(f32))
    noz = jnp.zeros((t_pad, R), f32).at[:T].set(
        (bdt_h * noise_seq.reshape(T, R)).astype(f32))
    steps = jnp.concatenate([stim, noz], axis=1)

    init_state = jnp.concatenate(
        [delay_chain.astype(f32).reshape(1, DS),
         eligibility_trace.astype(f32).reshape(1, DS),
         C.astype(f32).reshape(1, R)], axis=1)

    kernel_fn = functools.partial(
        _pro_chunk_kernel,
        n_stim=S, n_resp=R, n_ro=RO, n_delay=n_delay, hidden=H,
        chunk=chunk, last_chunk=last_chunk, last_local=last_local,
        lam_chunk=float(lam ** chunk), lam_last=float(lam ** (last_local + 1)),
        dt=float(hp["dt"]), beta=float(hp["beta"]),
        psi=float(hp["psi"]), phi=float(hp["phi"]), rho=float(hp["rho"]),
        response_threshold=float(hp["response_threshold"]),
    )

    def const_spec(shape):
        return pl.BlockSpec(shape, lambda c: (0,) * len(shape))

    per_step_out, final_state = pl.pallas_call(
        kernel_fn,
        grid=(num_chunks,),
        in_specs=[
            pl.BlockSpec((chunk, S + R), lambda c: (c, 0)),
            const_spec((1, state_len)),
            const_spec((chunk, chunk)),
            const_spec((8, chunk)),
            const_spec((chunk, 1)),
            const_spec(tuple(w1.shape)),
            const_spec(tuple(w2.shape)),
            const_spec(tuple(u2d.shape)),
            const_spec(tuple(w3.shape)),
            const_spec(tuple(w_i.shape)),
            const_spec(tuple(wb.shape)),
            const_spec(tuple(bias.shape)),
        ],
        out_specs=(
            pl.BlockSpec((chunk, OUT_W), lambda c: (c, 0)),
            const_spec((1, state_len)),
        ),
        out_shape=(
            jax.ShapeDtypeStruct((t_pad, OUT_W), f32),
            jax.ShapeDtypeStruct((1, state_len), f32),
        ),
        scratch_shapes=[
            pltpu.VMEM((1, state_len), f32),
            pltpu.VMEM((chunk + 16, S), f32),
            pltpu.VMEM((chunk + 16, S), jnp.bfloat16),
            pltpu.VMEM((16 + chunk // 8 + 8, 8 * R), f32),
            pltpu.VMEM((16, 2 * R), f32),
        ],
        compiler_params=pltpu.CompilerParams(
            dimension_semantics=("arbitrary",)),
    )(steps, init_state, lam_mat, l_rows, lam_pows, w1, w2, u2d, w3, w_i, wb, bias)

    rows = per_step_out[:T]
    resp = rows[:, 0:R]
    disc = rows[:, R:2 * R]
    ro_pred = rows[:, 2 * R:2 * R + RO]
    temporal = rows[:, 2 * R + RO:2 * R + 2 * RO]
    fs = final_state[0]
    new_state = dict(
        delay_chain=fs[0:DS].reshape(n_delay, S),
        eligibility_trace=fs[DS:2 * DS].reshape(n_delay, S),
        C=fs[2 * DS:2 * DS + R],
    )
    return resp, ro_pred, temporal, disc, new_state


# speedup vs baseline: 1.0218x; 1.0218x over previous
import functools

import jax
import jax.numpy as jnp
from jax import lax
from jax.experimental import pallas as pl
from jax.experimental.pallas import tpu as pltpu

_NSWEEP = 12
_LAG = 4


def _pro_chunk_kernel(
    steps_ref,
    init_ref,
    l_ref,
    lrows_ref,
    lam_ref,
    w1_ref,
    w2_ref,
    u2d_ref,
    w3_ref,
    wi_ref,
    wb_ref,
    b_ref,
    out_ref,
    state_out_ref,
    state_s,
    e_s,
    eb_s,
    pk_s,
    st_s,
    *,
    n_stim, n_resp, n_ro, n_delay, hidden,
    chunk, last_chunk, last_local, lam_chunk, lam_last,
    dt, beta, psi, phi, rho, response_threshold,
):
    f32 = jnp.float32
    bf16 = jnp.bfloat16
    S, R, RO, H = n_stim, n_resp, n_ro, hidden
    DS = n_delay * S
    CH = chunk
    cid = pl.program_id(0)

    @pl.when(cid == 0)
    def _():
        state_s[...] = init_ref[...]

    stim = steps_ref[:, 0:S]
    noise = steps_ref[:, S:S + R]

    OFF = 16
    for i in range(n_delay - 1):
        lo = (n_delay - 2 - i) * S
        row = state_s[:, lo:lo + S]
        e_s[OFF - (n_delay - 1) + i:OFF - (n_delay - 1) + i + 1, :] = row
        eb_s[OFF - (n_delay - 1) + i:OFF - (n_delay - 1) + i + 1, :] = row.astype(bf16)
    e_s[OFF:OFF + CH, :] = stim
    eb_s[OFF:OFF + CH, :] = stim.astype(bf16)
    d_mat_b = jnp.concatenate(
        [eb_s[OFF - d:OFF - d + CH, :] for d in range(n_delay)],
        axis=1)

    carry = state_s[:, DS:2 * DS]
    du = jnp.dot(d_mat_b, u2d_ref[...], preferred_element_type=f32)
    cu = jnp.dot(carry.astype(bf16), u2d_ref[...],
                 preferred_element_type=f32)
    temporal = (jnp.dot(l_ref[...], du, preferred_element_type=f32)
                + lam_ref[...] * cu)
    er_dot = jnp.dot(lrows_ref[...], d_mat_b,
                     preferred_element_type=f32)

    fc1_b = b_ref[:, 0:H]
    fc2_b = b_ref[:, H:H + RO]

    res1 = jnp.dot(stim, w1_ref[...], preferred_element_type=f32)
    ws_out = res1[:, 0:RO]
    wc_out = res1[:, RO:RO + R]
    h = jnp.maximum(res1[:, RO + R:RO + R + H] + fc1_b, 0.0)

    valence = jnp.dot(h, w2_ref[...], preferred_element_type=f32) + fc2_b
    ro_pred = ws_out * valence

    lhs3 = jnp.concatenate([ro_pred, jnp.maximum(ro_pred, 0.0)], axis=1)
    res3 = jnp.dot(lhs3, w3_ref[...], preferred_element_type=f32)

    exc = wc_out
    ctb = jnp.maximum(res3[:, R:2 * R], 0.0) - res3[:, 0:R]
    r_vec = exc + noise - 0.05 * ctb
    sm1 = 1.0 - exc - ctb

    G = 8
    W = CH // G
    K = 16
    P = K + W
    c0 = state_s[:, 2 * DS:2 * DS + R]
    w_i = wi_ref[...]

    @pl.when(cid == 0)
    def _():
        inh0 = jnp.dot(c0, w_i, preferred_element_type=f32)
        z0 = jnp.log(c0 / (1.0 - c0))
        st_s[:, 0:R] = jnp.broadcast_to(z0 + (c0 + 0.05) * inh0, (K, R))
        st_s[:, R:2 * R] = jnp.zeros((K, R), f32)

    stash_r = st_s[:, 0:R]
    stash_s = st_s[:, R:2 * R]
    cols_r, cols_s = [], []
    for g in range(G):
        lo = g * W - K
        if lo < 0:
            cols_r.append(jnp.concatenate(
                [stash_r[lo + K:K, :], r_vec[0:g * W + W, :]], axis=0))
            cols_s.append(jnp.concatenate(
                [stash_s[lo + K:K, :], sm1[0:g * W + W, :]], axis=0))
        else:
            cols_r.append(r_vec[lo:lo + P, :])
            cols_s.append(sm1[lo:lo + P, :])
    rp_pk = jnp.concatenate(cols_r, axis=1)
    sm_pk = jnp.concatenate(cols_s, axis=1)

    pk_s[0:1, :] = jnp.concatenate([c0] * G, axis=1)

    @pl.when(cid == 0)
    def _():
        pk_s[1:P + 1, :] = jnp.broadcast_to(
            jnp.concatenate([c0] * G, axis=1), (P, G * R))

    wb = wb_ref[...]
    inh_q = []
    for m in range(_NSWEEP):
        src = pk_s[0:P, :]
        if m <= _NSWEEP - 1 - _LAG or m == 0:
            inh_q.append(jnp.dot(src, wb, preferred_element_type=f32))
        inh = inh_q[max(0, m - _LAG)]
        pre = src * sm_pk + rp_pk - (src + 0.05) * inh
        pk_s[1:P + 1, :] = 0.5 * jnp.tanh(0.5 * pre) + 0.5

    traj = pk_s[K + 1:P + 1, :]
    c_all = jnp.concatenate(
        [traj[:, g * R:(g + 1) * R] for g in range(G)], axis=0)

    st_s[:, 0:R] = r_vec[CH - K:CH, :]
    st_s[:, R:2 * R] = sm1[CH - K:CH, :]

    out_ref[:, 0:R] = c_all
    out_ref[:, R:2 * R] = (c_all > response_threshold).astype(f32)
    out_ref[:, 2 * R:2 * R + RO] = ro_pred
    out_ref[:, 2 * R + RO:2 * R + 2 * RO] = temporal
    out_ref[:, 2 * R + 2 * RO:] = jnp.zeros(
        (CH, out_ref.shape[1] - (2 * R + 2 * RO)), f32)

    d_last = jnp.concatenate(
        [e_s[OFF - d + CH - 1:OFF - d + CH, :]
         for d in range(n_delay)], axis=1)
    state_s[:, 0:DS] = d_last
    state_s[:, DS:2 * DS] = er_dot[0:1, :] + lam_chunk * carry
    state_s[:, 2 * DS:2 * DS + R] = c_all[CH - 1:CH, :]

    @pl.when(cid == last_chunk)
    def _():
        state_out_ref[:, 0:DS] = jnp.concatenate(
            [e_s[OFF - d + last_local:OFF - d + last_local + 1, :]
             for d in range(n_delay)], axis=1)
        state_out_ref[:, DS:2 * DS] = er_dot[1:2, :] + lam_last * carry
        state_out_ref[:, 2 * DS:2 * DS + R] = c_all[last_local:last_local + 1, :]


def kernel(W_S, W_C, W_F, W_R, W_I, U, fc1_w, fc1_b, fc2_w, fc2_b,
           delay_chain, eligibility_trace, C, stimuli_seq, noise_seq):
    f32 = jnp.float32
    bf16 = jnp.bfloat16
    hp = dict(dt=0.1, beta=0.1, lambda_decay=0.95, psi=0.1, phi=0.1, rho=0.1,
              response_threshold=0.5)
    n_ro, n_stim = W_S.shape
    n_resp = W_C.shape[0]
    n_delay = delay_chain.shape[0]
    hidden = fc1_b.shape[0]
    S, R, RO, H = n_stim, n_resp, n_ro, hidden
    DS = n_delay * S
    state_len = 2 * DS + R
    T = int(stimuli_seq.shape[0])
    OUT_W = 128
    assert 2 * R + 2 * RO <= OUT_W

    max_chunk = 1024
    chunk = min(max_chunk, ((T + 7) // 8) * 8)
    t_pad = ((T + chunk - 1) // chunk) * chunk
    num_chunks = t_pad // chunk
    last_chunk = (T - 1) // chunk
    last_local = (T - 1) % chunk

    lam = float(hp["lambda_decay"])
    idx = jnp.arange(chunk)
    diff = idx[:, None] - idx[None, :]
    lam_mat = jnp.where(
        diff >= 0,
        jnp.power(jnp.float32(lam), jnp.maximum(diff, 0).astype(f32)),
        0.0).astype(f32)
    l_rows = jnp.zeros((8, chunk), f32)
    l_rows = l_rows.at[0].set(lam_mat[chunk - 1])
    l_rows = l_rows.at[1].set(lam_mat[last_local])
    l_rows = l_rows.astype(bf16)
    lam_pows = jnp.power(jnp.float32(lam),
                         (idx + 1).astype(f32)).reshape(chunk, 1)

    bdt_h = float(hp["beta"]) * float(hp["dt"])
    w1 = jnp.zeros((S, RO + R + H), f32)
    w1 = w1.at[:, :RO].set(W_S.T.astype(f32))
    w1 = w1.at[:, RO:RO + R].set((bdt_h * float(hp["rho"])) * W_C.T.astype(f32))
    w1 = w1.at[:, RO + R:].set(fc1_w.T.astype(f32))

    w2 = fc2_w.T.astype(f32)
    u2d = U.astype(f32).reshape(RO, DS).T.astype(bf16)

    w3 = jnp.zeros((2 * RO, 2 * R), f32)
    w3 = w3.at[:RO, :R].set(W_F.astype(f32))
    w3 = w3.at[RO:2 * RO, R:2 * R].set(W_R.T.astype(f32))
    w3 = (bdt_h * float(hp["phi"])) * w3

    bdt = float(hp["beta"]) * float(hp["dt"])
    w_i = (bdt * float(hp["psi"])) * W_I.astype(f32)
    wb = jnp.kron(jnp.eye(8, dtype=f32), w_i)
    bias = jnp.concatenate([fc1_b.astype(f32),
                            fc2_b.astype(f32)]).reshape(1, H + RO)

    stim = jnp.zeros((t_pad, S), f32).at[:T].set(
        stimuli_seq.reshape(T, S).astype(f32))
    noz = jnp.zeros((t_pad, R), f32).at[:T].set(
        (bdt_h * noise_seq.reshape(T, R)).astype(f32))
    steps = jnp.concatenate([stim, noz], axis=1)

    init_state = jnp.concatenate(
        [delay_chain.astype(f32).reshape(1, DS),
         eligibility_trace.astype(f32).reshape(1, DS),
         C.astype(f32).reshape(1, R)], axis=1)

    kernel_fn = functools.partial(
        _pro_chunk_kernel,
        n_stim=S, n_resp=R, n_ro=RO, n_delay=n_delay, hidden=H,
        chunk=chunk, last_chunk=last_chunk, last_local=last_local,
        lam_chunk=float(lam ** chunk), lam_last=float(lam ** (last_local + 1)),
        dt=float(hp["dt"]), beta=float(hp["beta"]),
        psi=float(hp["psi"]), phi=float(hp["phi"]), rho=float(hp["rho"]),
        response_threshold=float(hp["response_threshold"]),
    )

    def const_spec(shape):
        return pl.BlockSpec(shape, lambda c: (0,) * len(shape))

    per_step_out, final_state = pl.pallas_call(
        kernel_fn,
        grid=(num_chunks,),
        in_specs=[
            pl.BlockSpec((chunk, S + R), lambda c: (c, 0)),
            const_spec((1, state_len)),
            const_spec((chunk, chunk)),
            const_spec((8, chunk)),
            const_spec((chunk, 1)),
            const_spec(tuple(w1.shape)),
            const_spec(tuple(w2.shape)),
            const_spec(tuple(u2d.shape)),
            const_spec(tuple(w3.shape)),
            const_spec(tuple(w_i.shape)),
            const_spec(tuple(wb.shape)),
            const_spec(tuple(bias.shape)),
        ],
        out_specs=(
            pl.BlockSpec((chunk, OUT_W), lambda c: (c, 0)),
            const_spec((1, state_len)),
        ),
        out_shape=(
            jax.ShapeDtypeStruct((t_pad, OUT_W), f32),
            jax.ShapeDtypeStruct((1, state_len), f32),
        ),
        scratch_shapes=[
            pltpu.VMEM((1, state_len), f32),
            pltpu.VMEM((chunk + 16, S), f32),
            pltpu.VMEM((chunk + 16, S), jnp.bfloat16),
            pltpu.VMEM((16 + chunk // 8 + 8, 8 * R), f32),
            pltpu.VMEM((16, 2 * R), f32),
        ],
        compiler_params=pltpu.CompilerParams(
            dimension_semantics=("arbitrary",)),
    )(steps, init_state, lam_mat, l_rows, lam_pows, w1, w2, u2d, w3, w_i, wb, bias)

    rows = per_step_out[:T]
    resp = rows[:, 0:R]
    disc = rows[:, R:2 * R]
    ro_pred = rows[:, 2 * R:2 * R + RO]
    temporal = rows[:, 2 * R + RO:2 * R + 2 * RO]
    fs = final_state[0]
    new_state = dict(
        delay_chain=fs[0:DS].reshape(n_delay, S),
        eligibility_trace=fs[DS:2 * DS].reshape(n_delay, S),
        C=fs[2 * DS:2 * DS + R],
    )
    return resp, ro_pred, temporal, disc, new_state
